# Initial kernel scaffold; baseline (speedup 1.0000x reference)
import jax
import jax.numpy as jnp
from jax.experimental import pallas as pl


def kernel(x, edge_index, batch, w1_1, b1_1, w1_2, b1_2, bn1_g, bn1_b, bn1_m, bn1_v, w2_1, b2_1, w2_2, b2_2, bn2_g, bn2_b, bn2_m, bn2_v, w3_1, b3_1, w3_2, b3_2, bn3_g, bn3_b, bn3_m, bn3_v, wb, bb, wm, bm):
    raise NotImplementedError("write your pallas kernel here")


# speedup vs baseline: 7.2906x; 7.2906x over previous
import functools

import jax
import jax.numpy as jnp
from jax import lax
from jax.experimental import pallas as pl
from jax.experimental.pallas import tpu as pltpu
from jax.experimental.pallas import tpu_sc as plsc

N = 10000
E = 320000
IN_DIM = 128
H = 32
G = 256
DEC = 16

NW = 32
CH = 128
NCH = 80
EPT = NCH * CH
EPAD = NW * EPT
RPS = 632
AGN = 16 * RPS
NPAD = AGN
BLK = RPS
NB = 16


def _proj_body(x_ref, w_ref, o_ref):
    o_ref[0:N, :] = jnp.dot(x_ref[...], w_ref[...],
                            preferred_element_type=jnp.float32)
    o_ref[N:NPAD, :] = jnp.zeros((NPAD - N, H), jnp.float32)


def _proj(x, w):
    return pl.pallas_call(
        _proj_body,
        out_shape=jax.ShapeDtypeStruct((NPAD, H), jnp.float32),
    )(x, w)


def _mid_body(y_ref, a_ref, b_ref, W_ref, c_ref, o_ref):
    t = jnp.maximum(
        y_ref[0:N, :] + a_ref[0:N, :] + a_ref[AGN:AGN + N, :] + b_ref[...],
        0.0)
    o_ref[0:N, :] = jnp.dot(t, W_ref[...],
                            preferred_element_type=jnp.float32) + c_ref[...]
    o_ref[N:NPAD, :] = jnp.zeros((NPAD - N, H), jnp.float32)


def _mid(y, a, b, W, c):
    return pl.pallas_call(
        _mid_body,
        out_shape=jax.ShapeDtypeStruct((NPAD, H), jnp.float32),
    )(y, a, b, W, c)


def _tail_body(y_ref, aA_ref, aB_ref, bat_ref, b_ref, W4_ref, c4_ref,
               bb_ref, wm_ref, bm_ref, o_ref, sums_s, cnt_s):
    j = pl.program_id(0)

    @pl.when(j == 0)
    def _():
        sums_s[...] = jnp.zeros_like(sums_s)
        cnt_s[...] = jnp.zeros_like(cnt_s)

    @pl.when(j < NB)
    def _():
        t = jnp.maximum(
            y_ref[...] + aA_ref[...] + aB_ref[...] + b_ref[...], 0.0)
        bat = bat_ref[...].reshape(1, BLK)
        oh = (lax.broadcasted_iota(jnp.int32, (G, BLK), 0) == bat
              ).astype(jnp.float32)
        sums_s[...] = sums_s[...] + jnp.dot(
            oh, t, preferred_element_type=jnp.float32)
        cnt_s[...] = cnt_s[...] + jnp.sum(oh, axis=1, keepdims=True)

    @pl.when(j == NB)
    def _():
        cnt = cnt_s[...]
        mean = sums_s[...] / jnp.maximum(cnt, 1.0)
        zpre = jnp.where(
            cnt > 0.0,
            jnp.dot(mean, W4_ref[...],
                    preferred_element_type=jnp.float32) + c4_ref[...],
            bb_ref[...])
        z = jnp.maximum(zpre, 0.0)
        logit = jnp.dot(z, wm_ref[...],
                        preferred_element_type=jnp.float32) + bm_ref[...]
        o_ref[...] = 1.0 / (1.0 + jnp.exp(-logit))


def _tail(y, a, bat3, b, W4, c4, bb, wm, bm):
    idx = lambda j: (jnp.minimum(j, NB - 1), 0)
    return pl.pallas_call(
        _tail_body,
        grid=(NB + 1,),
        in_specs=[
            pl.BlockSpec((BLK, H), idx),
            pl.BlockSpec((BLK, H), idx),
            pl.BlockSpec((BLK, H), lambda j: (NB + jnp.minimum(j, NB - 1), 0)),
            pl.BlockSpec((1, 1, BLK), lambda j: (jnp.minimum(j, NB - 1), 0, 0)),
            pl.BlockSpec((1, H), lambda j: (0, 0)),
            pl.BlockSpec((H, DEC), lambda j: (0, 0)),
            pl.BlockSpec((1, DEC), lambda j: (0, 0)),
            pl.BlockSpec((1, DEC), lambda j: (0, 0)),
            pl.BlockSpec((DEC, 1), lambda j: (0, 0)),
            pl.BlockSpec((1, 1), lambda j: (0, 0)),
        ],
        out_specs=pl.BlockSpec((G, 1), lambda j: (0, 0)),
        scratch_shapes=[
            pltpu.VMEM((G, H), jnp.float32),
            pltpu.VMEM((G, 1), jnp.float32),
        ],
        out_shape=jax.ShapeDtypeStruct((G, 1), jnp.float32),
    )(y, a, a, bat3, b, W4, c4, bb, wm, bm)


@functools.partial(
    pl.kernel,
    out_type=jax.ShapeDtypeStruct((2 * AGN, H), jnp.float32),
    mesh=plsc.VectorSubcoreMesh(core_axis_name="c", subcore_axis_name="s"),
    compiler_params=pltpu.CompilerParams(use_tc_tiling_on_sc=False),
    scratch_types=[
        pltpu.VMEM((NCH, CH), jnp.int32),
        pltpu.VMEM((NCH, CH), jnp.int32),
        pltpu.VMEM((CH, H), jnp.float32),
        pltpu.VMEM_SHARED((AGN, H), jnp.float32),
        pltpu.SemaphoreType.DMA,
    ],
)
def _sc_agg(y_hbm, srcp_hbm, dstp_hbm, z_hbm, out_hbm,
            src_v, dst_v, rows_v, agg_sh, sem):
    c = lax.axis_index("c")
    s = lax.axis_index("s")
    tid = c * 16 + s
    pltpu.sync_copy(z_hbm, agg_sh.at[pl.ds(s * RPS, RPS)])
    pltpu.sync_copy(srcp_hbm.at[pl.ds(tid * NCH, NCH)], src_v)
    pltpu.sync_copy(dstp_hbm.at[pl.ds(tid * NCH, NCH)], dst_v)
    plsc.subcore_barrier()

    def _chunk(j, carry):
        pltpu.async_copy(y_hbm.at[src_v.at[j]], rows_v, sem).wait()
        pltpu.sync_copy(rows_v, agg_sh.at[dst_v.at[j]], add=True)
        return carry

    lax.fori_loop(0, NCH, _chunk, 0)
    plsc.subcore_barrier()
    pltpu.sync_copy(agg_sh.at[pl.ds(s * RPS, RPS)],
                    out_hbm.at[pl.ds(c * AGN + s * RPS, RPS)])


def kernel(x, edge_index, batch,
           w1_1, b1_1, w1_2, b1_2, bn1_g, bn1_b, bn1_m, bn1_v,
           w2_1, b2_1, w2_2, b2_2, bn2_g, bn2_b, bn2_m, bn2_v,
           w3_1, b3_1, w3_2, b3_2, bn3_g, bn3_b, bn3_m, bn3_v,
           wb, bb, wm, bm):
    s1 = bn1_g * lax.rsqrt(bn1_v + 1e-5)
    s2 = bn2_g * lax.rsqrt(bn2_v + 1e-5)
    s3 = bn3_g * lax.rsqrt(bn3_v + 1e-5)
    W2 = (w1_2 * s1[None, :]) @ w2_1
    c2 = ((b1_2 * s1 + bn1_b - bn1_m * s1) @ w2_1).reshape(1, H)
    W3 = (w2_2 * s2[None, :]) @ w3_1
    c3 = ((b2_2 * s2 + bn2_b - bn2_m * s2) @ w3_1).reshape(1, H)
    W4 = (w3_2 * s3[None, :]) @ wb
    c4 = ((b3_2 * s3 + bn3_b - bn3_m * s3) @ wb + bb).reshape(1, DEC)

    b1r = b1_1.reshape(1, H)
    b2r = b2_1.reshape(1, H)
    b3r = b3_1.reshape(1, H)
    bbr = bb.reshape(1, DEC)
    bmr = bm.reshape(1, 1)

    pad = EPAD - E
    srcp = jnp.concatenate(
        [edge_index[0], jnp.full((pad,), N, jnp.int32)]).reshape(NW * NCH, CH)
    dstp = jnp.concatenate(
        [edge_index[1], jnp.zeros((pad,), jnp.int32)]).reshape(NW * NCH, CH)
    zrows = jnp.zeros((RPS, H), jnp.float32)
    bat3 = jnp.concatenate(
        [batch, jnp.full((NPAD - N,), G, jnp.int32)]).reshape(NB, 1, BLK)

    y1 = _proj(x, w1_1)
    a1 = _sc_agg(y1, srcp, dstp, zrows)
    y2 = _mid(y1, a1, b1r, W2, c2)
    a2 = _sc_agg(y2, srcp, dstp, zrows)
    y3 = _mid(y2, a2, b2r, W3, c3)
    a3 = _sc_agg(y3, srcp, dstp, zrows)
    return _tail(y3, a3, bat3, b3r, W4, c4, bbr, wm, bmr)


# speedup vs baseline: 9.0339x; 1.2391x over previous
import functools

import jax
import jax.numpy as jnp
from jax import lax
from jax.experimental import pallas as pl
from jax.experimental.pallas import tpu as pltpu
from jax.experimental.pallas import tpu_sc as plsc

N = 10000
E = 320000
IN_DIM = 128
H = 32
G = 256
DEC = 16

NW = 32
CH = 128
NCH = 80
EPT = NCH * CH
EPAD = NW * EPT
RPS = 632
AGN = 16 * RPS
NPAD = AGN
BLK = RPS
NB = 16


def _proj_body(x_ref, w_ref, o_ref):
    o_ref[0:N, :] = jnp.dot(x_ref[...], w_ref[...],
                            preferred_element_type=jnp.float32)
    o_ref[N:NPAD, :] = jnp.zeros((NPAD - N, H), jnp.float32)


def _proj(x, w):
    return pl.pallas_call(
        _proj_body,
        out_shape=jax.ShapeDtypeStruct((NPAD, H), jnp.float32),
    )(x, w)


def _mid_body(y_ref, a_ref, b_ref, W_ref, c_ref, o_ref):
    t = jnp.maximum(
        y_ref[0:N, :] + a_ref[0:N, :] + a_ref[AGN:AGN + N, :] + b_ref[...],
        0.0)
    o_ref[0:N, :] = jnp.dot(t, W_ref[...],
                            preferred_element_type=jnp.float32) + c_ref[...]
    o_ref[N:NPAD, :] = jnp.zeros((NPAD - N, H), jnp.float32)


def _mid(y, a, b, W, c):
    return pl.pallas_call(
        _mid_body,
        out_shape=jax.ShapeDtypeStruct((NPAD, H), jnp.float32),
    )(y, a, b, W, c)


def _tail_body(y_ref, aA_ref, aB_ref, bat_ref, b_ref, W4_ref, c4_ref,
               bb_ref, wm_ref, bm_ref, o_ref, sums_s, cnt_s):
    j = pl.program_id(0)

    @pl.when(j == 0)
    def _():
        sums_s[...] = jnp.zeros_like(sums_s)
        cnt_s[...] = jnp.zeros_like(cnt_s)

    @pl.when(j < NB)
    def _():
        t = jnp.maximum(
            y_ref[...] + aA_ref[...] + aB_ref[...] + b_ref[...], 0.0)
        bat = bat_ref[...].reshape(1, BLK)
        oh = (lax.broadcasted_iota(jnp.int32, (G, BLK), 0) == bat
              ).astype(jnp.float32)
        sums_s[...] = sums_s[...] + jnp.dot(
            oh, t, preferred_element_type=jnp.float32)
        cnt_s[...] = cnt_s[...] + jnp.sum(oh, axis=1, keepdims=True)

    @pl.when(j == NB)
    def _():
        cnt = cnt_s[...]
        mean = sums_s[...] / jnp.maximum(cnt, 1.0)
        zpre = jnp.where(
            cnt > 0.0,
            jnp.dot(mean, W4_ref[...],
                    preferred_element_type=jnp.float32) + c4_ref[...],
            bb_ref[...])
        z = jnp.maximum(zpre, 0.0)
        logit = jnp.dot(z, wm_ref[...],
                        preferred_element_type=jnp.float32) + bm_ref[...]
        o_ref[...] = 1.0 / (1.0 + jnp.exp(-logit))


def _tail(y, a, bat3, b, W4, c4, bb, wm, bm):
    idx = lambda j: (jnp.minimum(j, NB - 1), 0)
    return pl.pallas_call(
        _tail_body,
        grid=(NB + 1,),
        in_specs=[
            pl.BlockSpec((BLK, H), idx),
            pl.BlockSpec((BLK, H), idx),
            pl.BlockSpec((BLK, H), lambda j: (NB + jnp.minimum(j, NB - 1), 0)),
            pl.BlockSpec((1, 1, BLK), lambda j: (jnp.minimum(j, NB - 1), 0, 0)),
            pl.BlockSpec((1, H), lambda j: (0, 0)),
            pl.BlockSpec((H, DEC), lambda j: (0, 0)),
            pl.BlockSpec((1, DEC), lambda j: (0, 0)),
            pl.BlockSpec((1, DEC), lambda j: (0, 0)),
            pl.BlockSpec((DEC, 1), lambda j: (0, 0)),
            pl.BlockSpec((1, 1), lambda j: (0, 0)),
        ],
        out_specs=pl.BlockSpec((G, 1), lambda j: (0, 0)),
        scratch_shapes=[
            pltpu.VMEM((G, H), jnp.float32),
            pltpu.VMEM((G, 1), jnp.float32),
        ],
        out_shape=jax.ShapeDtypeStruct((G, 1), jnp.float32),
    )(y, a, a, bat3, b, W4, c4, bb, wm, bm)


K = 8
NG = NCH // K


@functools.partial(
    pl.kernel,
    out_type=jax.ShapeDtypeStruct((2 * AGN, H), jnp.float32),
    mesh=plsc.VectorSubcoreMesh(core_axis_name="c", subcore_axis_name="s"),
    compiler_params=pltpu.CompilerParams(use_tc_tiling_on_sc=False),
    scratch_types=[
        pltpu.VMEM((NCH, CH), jnp.int32),
        pltpu.VMEM((NCH, CH), jnp.int32),
        pltpu.VMEM((2, K, CH, H), jnp.float32),
        pltpu.VMEM_SHARED((AGN, H), jnp.float32),
        pltpu.SemaphoreType.DMA,
        pltpu.SemaphoreType.DMA,
    ],
)
def _sc_agg(y_hbm, srcp_hbm, dstp_hbm, z_hbm, out_hbm,
            src_v, dst_v, rows_v, agg_sh, sem_g, sem_s):
    c = lax.axis_index("c")
    s = lax.axis_index("s")
    tid = c * 16 + s
    pltpu.sync_copy(z_hbm, agg_sh.at[pl.ds(s * RPS, RPS)])
    pltpu.sync_copy(srcp_hbm.at[pl.ds(tid * NCH, NCH)], src_v)
    pltpu.sync_copy(dstp_hbm.at[pl.ds(tid * NCH, NCH)], dst_v)
    plsc.subcore_barrier()

    for k in range(K):
        pltpu.async_copy(y_hbm.at[src_v.at[k]], rows_v.at[0, k], sem_g)

    def _drain(sem, half, k):
        pltpu.make_async_copy(y_hbm.at[pl.ds(0, CH)],
                              rows_v.at[half, k], sem).wait()

    def _group(g, carry):
        half = lax.rem(g, 2)
        other = 1 - half
        for k in range(K):
            _drain(sem_g, half, k)

        @pl.when(g >= 1)
        def _():
            for k in range(K):
                _drain(sem_s, other, k)

        @pl.when(g + 1 < NG)
        def _():
            for k in range(K):
                pltpu.async_copy(y_hbm.at[src_v.at[(g + 1) * K + k]],
                                 rows_v.at[other, k], sem_g)

        for k in range(K):
            pltpu.async_copy(rows_v.at[half, k],
                             agg_sh.at[dst_v.at[g * K + k]], sem_s, add=True)
        return carry

    lax.fori_loop(0, NG, _group, 0)
    for k in range(K):
        _drain(sem_s, (NG - 1) % 2, k)
    plsc.subcore_barrier()
    pltpu.sync_copy(agg_sh.at[pl.ds(s * RPS, RPS)],
                    out_hbm.at[pl.ds(c * AGN + s * RPS, RPS)])


def kernel(x, edge_index, batch,
           w1_1, b1_1, w1_2, b1_2, bn1_g, bn1_b, bn1_m, bn1_v,
           w2_1, b2_1, w2_2, b2_2, bn2_g, bn2_b, bn2_m, bn2_v,
           w3_1, b3_1, w3_2, b3_2, bn3_g, bn3_b, bn3_m, bn3_v,
           wb, bb, wm, bm):
    s1 = bn1_g * lax.rsqrt(bn1_v + 1e-5)
    s2 = bn2_g * lax.rsqrt(bn2_v + 1e-5)
    s3 = bn3_g * lax.rsqrt(bn3_v + 1e-5)
    W2 = (w1_2 * s1[None, :]) @ w2_1
    c2 = ((b1_2 * s1 + bn1_b - bn1_m * s1) @ w2_1).reshape(1, H)
    W3 = (w2_2 * s2[None, :]) @ w3_1
    c3 = ((b2_2 * s2 + bn2_b - bn2_m * s2) @ w3_1).reshape(1, H)
    W4 = (w3_2 * s3[None, :]) @ wb
    c4 = ((b3_2 * s3 + bn3_b - bn3_m * s3) @ wb + bb).reshape(1, DEC)

    b1r = b1_1.reshape(1, H)
    b2r = b2_1.reshape(1, H)
    b3r = b3_1.reshape(1, H)
    bbr = bb.reshape(1, DEC)
    bmr = bm.reshape(1, 1)

    pad = EPAD - E
    srcp = jnp.concatenate(
        [edge_index[0], jnp.full((pad,), N, jnp.int32)]).reshape(NW * NCH, CH)
    dstp = jnp.concatenate(
        [edge_index[1], jnp.zeros((pad,), jnp.int32)]).reshape(NW * NCH, CH)
    zrows = jnp.zeros((RPS, H), jnp.float32)
    bat3 = jnp.concatenate(
        [batch, jnp.full((NPAD - N,), G, jnp.int32)]).reshape(NB, 1, BLK)

    y1 = _proj(x, w1_1)
    a1 = _sc_agg(y1, srcp, dstp, zrows)
    y2 = _mid(y1, a1, b1r, W2, c2)
    a2 = _sc_agg(y2, srcp, dstp, zrows)
    y3 = _mid(y2, a2, b2r, W3, c3)
    a3 = _sc_agg(y3, srcp, dstp, zrows)
    return _tail(y3, a3, bat3, b3r, W4, c4, bbr, wm, bmr)
